# baseline (speedup 1.0000x reference)
import jax
import jax.numpy as jnp
from jax import lax
from jax.experimental import pallas as pl
from jax.experimental.pallas import tpu as pltpu
from jax.experimental.pallas import tpu_sc as plsc

NUM_SIGNALS = 1000000
CODE_DIM = 32
BATCH = 16384

_PACK = 128 // CODE_DIM
_NLINES = NUM_SIGNALS // _PACK

_NC = 2
_NS = 16
_NW = _NC * _NS
_BPW = BATCH // _NW
_CHUNK = 128
_NCHUNK = _BPW // _CHUNK
_WPW = _BPW * CODE_DIM
_NVREG = _WPW // 16


def _gather_body(line_idx_hbm, colid_hbm, table_hbm, out_hbm,
                 idx_v, colid_v, rows_v, out_v, sem):
    wid = lax.axis_index("s") * _NC + lax.axis_index("c")
    pltpu.sync_copy(line_idx_hbm.at[wid], idx_v)
    pltpu.sync_copy(colid_hbm.at[wid], colid_v)
    copies = [
        pltpu.async_copy(
            table_hbm.at[idx_v.at[j]],
            rows_v.at[pl.ds(j * _CHUNK, _CHUNK)],
            sem,
        )
        for j in range(_NCHUNK)
    ]
    for c in copies:
        c.wait()

    def body(m, _):
        base = pl.multiple_of(m * 16, 16)
        rid = jnp.full((16,), m >> 1, dtype=jnp.int32)
        cid = colid_v[pl.ds(base, 16)]
        vals = plsc.load_gather(rows_v, [rid, cid])
        out_v[pl.ds(base, 16)] = vals
        return _

    lax.fori_loop(0, _NVREG, body, None)
    pltpu.sync_copy(out_v, out_hbm.at[wid])


_mesh = plsc.VectorSubcoreMesh(core_axis_name="c", subcore_axis_name="s")


@jax.jit
def _gather(line_idx, colid, table):
    return pl.kernel(
        _gather_body,
        mesh=_mesh,
        out_type=jax.ShapeDtypeStruct((_NW, _WPW), jnp.float32),
        scratch_types=[
            pltpu.VMEM((_NCHUNK, _CHUNK), jnp.int32),
            pltpu.VMEM((_WPW,), jnp.int32),
            pltpu.VMEM((_BPW, 128), jnp.float32),
            pltpu.VMEM((_WPW,), jnp.float32),
            pltpu.SemaphoreType.DMA,
        ],
        compiler_params=pltpu.CompilerParams(needs_layout_passes=False),
    )(line_idx, colid, table)


def kernel(signal_indices, codes):
    idx32 = signal_indices.astype(jnp.int32)
    table = codes.reshape(_NLINES, 128)
    line_idx = (idx32 >> 2).reshape(_NW, _NCHUNK, _CHUNK)
    off = (idx32 & 3) << 5
    colid = (off[:, None] + jnp.arange(CODE_DIM, dtype=jnp.int32)[None, :])
    colid = colid.reshape(_NW, _WPW)
    out = _gather(line_idx, colid, table)
    return out.reshape(BATCH, CODE_DIM)


# speedup vs baseline: 1.0003x; 1.0003x over previous
import jax
import jax.numpy as jnp
from jax import lax
from jax.experimental import pallas as pl
from jax.experimental.pallas import tpu as pltpu
from jax.experimental.pallas import tpu_sc as plsc

NUM_SIGNALS = 1000000
CODE_DIM = 32
BATCH = 16384

_PACK = 128 // CODE_DIM
_NLINES = NUM_SIGNALS // _PACK

_NC = 2
_NS = 16
_NW = _NC * _NS
_BPW = BATCH // _NW
_CHUNK = 128
_NCHUNK = _BPW // _CHUNK
_WPW = _BPW * CODE_DIM
_NVREG = _WPW // 16


def _gather_body(line_idx_hbm, colid_hbm, table_hbm, out_hbm,
                 idx_v, colid_v, rows_v, out_v, sem):
    wid = lax.axis_index("s") * _NC + lax.axis_index("c")
    pltpu.sync_copy(line_idx_hbm.at[wid], idx_v)
    pltpu.sync_copy(colid_hbm.at[wid], colid_v)
    copies = [
        pltpu.async_copy(
            table_hbm.at[idx_v.at[j]],
            rows_v.at[pl.ds(j * _CHUNK, _CHUNK)],
            sem,
        )
        for j in range(_NCHUNK)
    ]
    for c in copies:
        c.wait()

    def body(m, _):
        base = pl.multiple_of(m * 16, 16)
        rid = jnp.full((16,), m >> 1, dtype=jnp.int32)
        cid = colid_v[pl.ds(base, 16)]
        vals = plsc.load_gather(rows_v, [rid, cid])
        out_v[pl.ds(base, 16)] = vals
        return _

    lax.fori_loop(0, _NVREG, body, None)
    pltpu.sync_copy(out_v, out_hbm.at[wid])


_mesh = plsc.VectorSubcoreMesh(core_axis_name="c", subcore_axis_name="s")


@jax.jit
def _gather(line_idx, colid, table):
    return pl.kernel(
        _gather_body,
        mesh=_mesh,
        out_type=jax.ShapeDtypeStruct((_NW, _WPW), jnp.float32),
        scratch_types=[
            pltpu.VMEM((_NCHUNK, _CHUNK), jnp.int32),
            pltpu.VMEM((_WPW,), jnp.int32),
            pltpu.VMEM((_BPW, 128), jnp.float32),
            pltpu.VMEM((_WPW,), jnp.float32),
            pltpu.SemaphoreType.DMA,
        ],
        compiler_params=pltpu.CompilerParams(
            needs_layout_passes=False, use_tc_tiling_on_sc=True),
    )(line_idx, colid, table)


def kernel(signal_indices, codes):
    idx32 = signal_indices.astype(jnp.int32)
    table = codes.reshape(_NLINES, 128)
    line_idx = (idx32 >> 2).reshape(_NW, _NCHUNK, _CHUNK)
    off = (idx32 & 3) << 5
    colid = (off[:, None] + jnp.arange(CODE_DIM, dtype=jnp.int32)[None, :])
    colid = colid.reshape(_NW, _WPW)
    out = _gather(line_idx, colid, table)
    return out.reshape(BATCH, CODE_DIM)


# speedup vs baseline: 1.0220x; 1.0217x over previous
import jax
import jax.numpy as jnp
from jax import lax
from jax.experimental import pallas as pl
from jax.experimental.pallas import tpu as pltpu
from jax.experimental.pallas import tpu_sc as plsc

NUM_SIGNALS = 1000000
CODE_DIM = 32
BATCH = 16384

_PACK = 128 // CODE_DIM
_NLINES = NUM_SIGNALS // _PACK

_NC = 2
_NS = 16
_NW = _NC * _NS
_BPW = BATCH // _NW
_CHUNK = 128
_NCHUNK = _BPW // _CHUNK


def _gather_body(idx_hbm, table_hbm, outT_hbm,
                 idx_v, line_v, off_v, rows_v, out_v, sem):
    wid = lax.axis_index("s") * _NC + lax.axis_index("c")
    base = wid * _BPW
    pltpu.sync_copy(idx_hbm.at[pl.ds(base, _BPW)], idx_v)
    for k in range(_BPW // 16):
        v = idx_v[pl.ds(k * 16, 16)]
        line_v[k // 8, pl.ds((k * 16) % _CHUNK, 16)] = v >> 2
        off_v[pl.ds(k * 16, 16)] = (v & 3) << 5
    copies = [
        pltpu.async_copy(
            table_hbm.at[line_v.at[j]],
            rows_v.at[pl.ds(j * _CHUNK, _CHUNK)],
            sem,
        )
        for j in range(_NCHUNK)
    ]
    for c in copies:
        c.wait()

    def extract(mb, _):
        s = pl.multiple_of(mb * 16, 16)
        bvec = lax.iota(jnp.int32, 16) + mb * 16
        offv = off_v[pl.ds(s, 16)]
        for c in range(CODE_DIM):
            vals = plsc.load_gather(rows_v, [bvec, offv + c])
            out_v[c, pl.ds(s, 16)] = vals
        return _

    lax.fori_loop(0, _BPW // 16, extract, None)
    pltpu.sync_copy(out_v, outT_hbm.at[:, pl.ds(base, _BPW)])


_mesh = plsc.VectorSubcoreMesh(core_axis_name="c", subcore_axis_name="s")


@jax.jit
def _gather(idx, table):
    return pl.kernel(
        _gather_body,
        mesh=_mesh,
        out_type=jax.ShapeDtypeStruct((CODE_DIM, BATCH), jnp.float32),
        scratch_types=[
            pltpu.VMEM((_BPW,), jnp.int32),
            pltpu.VMEM((_NCHUNK, _CHUNK), jnp.int32),
            pltpu.VMEM((_BPW,), jnp.int32),
            pltpu.VMEM((_BPW, 128), jnp.float32),
            pltpu.VMEM((CODE_DIM, _BPW), jnp.float32),
            pltpu.SemaphoreType.DMA,
        ],
        compiler_params=pltpu.CompilerParams(
            needs_layout_passes=False, use_tc_tiling_on_sc=True),
    )(idx, table)


def kernel(signal_indices, codes):
    idx = signal_indices.astype(jnp.int32)
    table = codes.reshape(_NLINES, 128)
    out_t = _gather(idx, table)
    return out_t.T
